# baseline (speedup 1.0000x reference)
import functools

import jax
import jax.numpy as jnp
from jax.experimental import pallas as pl
from jax.experimental.pallas import tpu as pltpu

_ALPHA = 0.2
_NEG_BIG = -1e30


def _proj_kernel(x_ref, w_ref, a_ref, haug_ref, fc_ref, fn_ref, mx_ref):
    i = pl.program_id(0)
    h = jnp.dot(x_ref[...], w_ref[...], preferred_element_type=jnp.float32)
    f = a_ref.shape[0] // 2
    fn = jnp.dot(h, a_ref[:f, :], preferred_element_type=jnp.float32)
    fn_ref[...] = fn
    fc_ref[...] = jnp.dot(h, a_ref[f:, :], preferred_element_type=jnp.float32)

    haug_ref[:, :f] = h.astype(jnp.bfloat16)
    lane = jax.lax.broadcasted_iota(jnp.int32, (h.shape[0], f), 1)
    haug_ref[:, f:] = jnp.where(lane == 0, 1.0, 0.0).astype(jnp.bfloat16)

    @pl.when(i == 0)
    def _init():
        mx_ref[...] = jnp.full_like(mx_ref, _NEG_BIG)

    mx_ref[...] = jnp.maximum(mx_ref[...], jnp.max(fn))


def _vec_kernel(fc_ref, fnr_ref, mx_ref, u_ref, up_ref, v_ref, vp_ref):
    mx = mx_ref[...]
    b = fc_ref[...] + mx
    t = jnp.maximum(b, _ALPHA * b)
    u_ref[...] = jnp.exp(b - t)
    up_ref[...] = jnp.exp(_ALPHA * b - t)
    d = fnr_ref[...] - mx
    v_ref[...] = jnp.exp(d)
    vp_ref[...] = jnp.exp(_ALPHA * d)


def _attn_kernel(adj_hbm, u_ref, up_ref, v_ref, vp_ref, haug_ref, out_ref,
                 buf_ref, sem, *, bm, nbuf, n_row):
    i = pl.program_id(0)

    def _copy(k, slot):
        return pltpu.make_async_copy(
            adj_hbm.at[pl.ds(k * bm, bm), :], buf_ref.at[slot], sem.at[slot])

    @pl.when(i == 0)
    def _prefetch():
        for k in range(min(nbuf - 1, n_row)):
            _copy(k, k).start()

    nxt = i + nbuf - 1

    @pl.when(nxt < n_row)
    def _start_next():
        _copy(nxt, nxt % nbuf).start()

    slot = i % nbuf
    _copy(i, slot).wait()

    f = out_ref.shape[1]
    p = (jnp.maximum(u_ref[...] * v_ref[...], up_ref[...] * vp_ref[...])
         * buf_ref[slot].astype(jnp.float32)).astype(jnp.bfloat16)
    zaug = jnp.dot(p, haug_ref[...], preferred_element_type=jnp.float32)
    z = zaug[:, :f]
    s = zaug[:, f:f + 1]
    z = z / jnp.where(s > 0, s, 1.0)
    out_ref[...] = jnp.where(z > 0, z, jnp.exp(jnp.minimum(z, 0.0)) - 1.0)


def _gat(x, adj, W, a, bm, bs):
    n, f_in = x.shape
    f_out = W.shape[1]
    n_row = n // bm

    haug, fc, fn, mx = pl.pallas_call(
        _proj_kernel,
        grid=(n_row,),
        in_specs=[
            pl.BlockSpec((bm, f_in), lambda i: (i, 0)),
            pl.BlockSpec((f_in, f_out), lambda i: (0, 0)),
            pl.BlockSpec((2 * f_out, 1), lambda i: (0, 0)),
        ],
        out_specs=[
            pl.BlockSpec((bm, 2 * f_out), lambda i: (i, 0)),
            pl.BlockSpec((bm, 1), lambda i: (i, 0)),
            pl.BlockSpec((bm, 1), lambda i: (i, 0)),
            pl.BlockSpec((1, 1), lambda i: (0, 0)),
        ],
        out_shape=[
            jax.ShapeDtypeStruct((n, 2 * f_out), jnp.bfloat16),
            jax.ShapeDtypeStruct((n, 1), jnp.float32),
            jax.ShapeDtypeStruct((n, 1), jnp.float32),
            jax.ShapeDtypeStruct((1, 1), jnp.float32),
        ],
        compiler_params=pltpu.CompilerParams(
            dimension_semantics=("arbitrary",)),
    )(x, W, a)

    fn_row = fn.reshape(1, n)

    u, up, v, vp = pl.pallas_call(
        _vec_kernel,
        out_shape=[
            jax.ShapeDtypeStruct((n, 1), jnp.float32),
            jax.ShapeDtypeStruct((n, 1), jnp.float32),
            jax.ShapeDtypeStruct((1, n), jnp.float32),
            jax.ShapeDtypeStruct((1, n), jnp.float32),
        ],
    )(fc, fn_row, mx)

    nbuf = 6
    out = pl.pallas_call(
        functools.partial(_attn_kernel, bm=bm, nbuf=nbuf, n_row=n_row),
        grid=(n_row,),
        in_specs=[
            pl.BlockSpec(memory_space=pl.ANY),
            pl.BlockSpec((bm, 1), lambda i: (i, 0)),
            pl.BlockSpec((bm, 1), lambda i: (i, 0)),
            pl.BlockSpec((1, n), lambda i: (0, 0)),
            pl.BlockSpec((1, n), lambda i: (0, 0)),
            pl.BlockSpec((n, 2 * f_out), lambda i: (0, 0)),
        ],
        out_specs=pl.BlockSpec((bm, f_out), lambda i: (i, 0)),
        out_shape=jax.ShapeDtypeStruct((n, f_out), jnp.float32),
        scratch_shapes=[
            pltpu.VMEM((nbuf, bm, n), jnp.int32),
            pltpu.SemaphoreType.DMA((nbuf,)),
        ],
        compiler_params=pltpu.CompilerParams(
            dimension_semantics=("arbitrary",)),
    )(adj, u, up, v, vp, haug)
    return out


def _pick_block(n, cap):
    best = 8
    for b in range(8, cap + 1, 8):
        if n % b == 0:
            best = b
    return best


def kernel(input, adj, W, a):
    n = input.shape[0]
    bm = _pick_block(n, 200)
    bs = bm if bm <= 104 else _pick_block(bm, 104)
    return _gat(input, adj, W, a, bm, bs)


# speedup vs baseline: 1.1200x; 1.1200x over previous
import functools

import jax
import jax.numpy as jnp
from jax.experimental import pallas as pl
from jax.experimental.pallas import tpu as pltpu

_ALPHA = 0.2
_NEG_BIG = -1e30


def _proj_kernel(x_ref, w_ref, a_ref, haug_ref, fc_ref, fn_ref, mx_ref):
    i = pl.program_id(0)
    h = jnp.dot(x_ref[...], w_ref[...], preferred_element_type=jnp.float32)
    f = a_ref.shape[0] // 2
    fn = jnp.dot(h, a_ref[:f, :], preferred_element_type=jnp.float32)
    fn_ref[...] = fn
    fc_ref[...] = jnp.dot(h, a_ref[f:, :], preferred_element_type=jnp.float32)

    haug_ref[:, :f] = h.astype(jnp.bfloat16)
    lane = jax.lax.broadcasted_iota(jnp.int32, (h.shape[0], f), 1)
    haug_ref[:, f:] = jnp.where(lane == 0, 1.0, 0.0).astype(jnp.bfloat16)

    @pl.when(i == 0)
    def _init():
        mx_ref[...] = jnp.full_like(mx_ref, _NEG_BIG)

    mx_ref[...] = jnp.maximum(mx_ref[...], jnp.max(fn))


def _vec_kernel(fc_ref, fnr_ref, mx_ref, u_ref, up_ref, v_ref, vp_ref):
    mx = mx_ref[...]
    b = fc_ref[...] + mx
    t = jnp.maximum(b, _ALPHA * b)
    u_ref[...] = jnp.exp(b - t)
    up_ref[...] = jnp.exp(_ALPHA * b - t)
    d = fnr_ref[...] - mx
    v_ref[...] = jnp.exp(d)
    vp_ref[...] = jnp.exp(_ALPHA * d)


def _attn_kernel(adj_ref, u_ref, up_ref, v_ref, vp_ref, haug_ref, out_ref, *,
                 bs):
    bm = adj_ref.shape[0]
    f = out_ref.shape[1]
    v = v_ref[...]
    vp = vp_ref[...]
    haug = haug_ref[...]
    for c in range(bm // bs):
        r = slice(c * bs, (c + 1) * bs)
        p = (jnp.maximum(u_ref[r, :] * v, up_ref[r, :] * vp)
             * adj_ref[r, :].astype(jnp.float32)).astype(jnp.bfloat16)
        zaug = jnp.dot(p, haug, preferred_element_type=jnp.float32)
        z = zaug[:, :f]
        s = zaug[:, f:f + 1]
        z = z / jnp.where(s > 0, s, 1.0)
        out_ref[r, :] = jnp.where(z > 0, z, jnp.exp(jnp.minimum(z, 0.0)) - 1.0)


def _gat(x, adj, W, a, bm, bs):
    n, f_in = x.shape
    f_out = W.shape[1]
    n_row = n // bm

    haug, fc, fn, mx = pl.pallas_call(
        _proj_kernel,
        grid=(n_row,),
        in_specs=[
            pl.BlockSpec((bm, f_in), lambda i: (i, 0)),
            pl.BlockSpec((f_in, f_out), lambda i: (0, 0)),
            pl.BlockSpec((2 * f_out, 1), lambda i: (0, 0)),
        ],
        out_specs=[
            pl.BlockSpec((bm, 2 * f_out), lambda i: (i, 0)),
            pl.BlockSpec((bm, 1), lambda i: (i, 0)),
            pl.BlockSpec((bm, 1), lambda i: (i, 0)),
            pl.BlockSpec((1, 1), lambda i: (0, 0)),
        ],
        out_shape=[
            jax.ShapeDtypeStruct((n, 2 * f_out), jnp.bfloat16),
            jax.ShapeDtypeStruct((n, 1), jnp.float32),
            jax.ShapeDtypeStruct((n, 1), jnp.float32),
            jax.ShapeDtypeStruct((1, 1), jnp.float32),
        ],
        compiler_params=pltpu.CompilerParams(
            dimension_semantics=("arbitrary",)),
    )(x, W, a)

    fn_row = fn.reshape(1, n)

    u, up, v, vp = pl.pallas_call(
        _vec_kernel,
        out_shape=[
            jax.ShapeDtypeStruct((n, 1), jnp.float32),
            jax.ShapeDtypeStruct((n, 1), jnp.float32),
            jax.ShapeDtypeStruct((1, n), jnp.float32),
            jax.ShapeDtypeStruct((1, n), jnp.float32),
        ],
    )(fc, fn_row, mx)

    out = pl.pallas_call(
        functools.partial(_attn_kernel, bs=bs),
        grid=(n_row,),
        in_specs=[
            pl.BlockSpec((bm, n), lambda i: (i, 0)),
            pl.BlockSpec((bm, 1), lambda i: (i, 0)),
            pl.BlockSpec((bm, 1), lambda i: (i, 0)),
            pl.BlockSpec((1, n), lambda i: (0, 0)),
            pl.BlockSpec((1, n), lambda i: (0, 0)),
            pl.BlockSpec((n, 2 * f_out), lambda i: (0, 0)),
        ],
        out_specs=pl.BlockSpec((bm, f_out), lambda i: (i, 0)),
        out_shape=jax.ShapeDtypeStruct((n, f_out), jnp.float32),
        compiler_params=pltpu.CompilerParams(
            dimension_semantics=("arbitrary",)),
    )(adj, u, up, v, vp, haug)
    return out


def _pick_block(n, cap):
    best = 8
    for b in range(8, cap + 1, 8):
        if n % b == 0:
            best = b
    return best


def kernel(input, adj, W, a):
    n = input.shape[0]
    bm = _pick_block(n, 400)
    bs = bm
    return _gat(input, adj, W, a, bm, bs)
